# baseline (speedup 1.0000x reference)
import functools

import jax
import jax.numpy as jnp
from jax import lax
from jax.experimental import pallas as pl
from jax.experimental.pallas import tpu as pltpu
from jax.experimental.pallas import tpu_sc as plsc

N = 10000
E = 320000
R = 8
B = 4
D = 128
NUM_LAYERS = 4
P = 8192

NC = 2
NS = 16
NW = NC * NS
EW = E // NW
CH = 128
NCHUNK = 79
EWP = NCHUNK * CH
RPT = N // NS
BN = 400
BP = 512
WCOLS = (R + 1) * D
NP = 3
PR = 3456
SPT = PR // NS
GR = 128
NPAD = NP * PR

_MESH = plsc.VectorSubcoreMesh(
    core_axis_name="c", subcore_axis_name="s", num_cores=NC, num_subcores=NS
)

_SQRT_HALF = 0.7071067811865476


def _gelu(x):
    return 0.5 * x * (1.0 + lax.erf(x * _SQRT_HALF))


def _ln(h, g, b, eps=1e-5):
    mu = jnp.mean(h, axis=-1, keepdims=True)
    var = jnp.mean((h - mu) ** 2, axis=-1, keepdims=True)
    return g * (h - mu) / jnp.sqrt(var + eps) + b


def _bf16_round(x):
    return x.astype(jnp.bfloat16).astype(jnp.float32)


def _wcat_body(att_ref, basis_ref, root_ref, rel_ref, out_ref):
    att = _bf16_round(att_ref[...])
    basis = _bf16_round(basis_ref[...])
    root = root_ref[...]
    rel = rel_ref[...]
    for i in range(NUM_LAYERS):
        relm = rel[i : i + 1, :]
        ex = jnp.exp(relm - jnp.max(relm, axis=-1, keepdims=True))
        attn = ex / jnp.sum(ex, axis=-1, keepdims=True)
        for r in range(R):
            w = None
            for b in range(B):
                srb = att[i, r : r + 1, b : b + 1] * attn[:, r : r + 1]
                term = srb * basis[i, b]
                w = term if w is None else w + term
            out_ref[i, :, r * D : (r + 1) * D] = w
        out_ref[i, :, R * D :] = root[i]


def _make_wcat(att, basis, root, rel):
    return pl.pallas_call(
        _wcat_body,
        out_shape=jax.ShapeDtypeStruct((NUM_LAYERS, D, WCOLS), jnp.float32),
    )(att, basis, root, rel)


def _mm_body(h_ref, w_ref, t_ref, r_ref):
    a = jnp.dot(h_ref[...], w_ref[...], preferred_element_type=jnp.float32)
    t_ref[...] = a[:, : R * D]
    r_ref[...] = a[:, R * D :]


def _mm(h, w):
    return pl.pallas_call(
        _mm_body,
        grid=(N // BN,),
        in_specs=[
            pl.BlockSpec((BN, D), lambda i: (i, 0)),
            pl.BlockSpec((D, WCOLS), lambda i: (0, 0)),
        ],
        out_specs=(
            pl.BlockSpec((BN, R * D), lambda i: (i, 0)),
            pl.BlockSpec((BN, D), lambda i: (i, 0)),
        ),
        out_shape=(
            jax.ShapeDtypeStruct((N, R * D), jnp.float32),
            jax.ShapeDtypeStruct((N, D), jnp.float32),
        ),
    )(h, w)


def _fused_body(part_ref, hroot_ref, hin_ref, w_ref, g_ref, b_ref,
                h_ref, t_ref, r_ref):
    agg = part_ref[0] + part_ref[1] + hroot_ref[...]
    h = _ln(agg, g_ref[...], b_ref[...]) + hin_ref[...]
    h = _gelu(h)
    h_ref[...] = h
    a = jnp.dot(h, w_ref[...], preferred_element_type=jnp.float32)
    t_ref[...] = a[:, : R * D]
    r_ref[...] = a[:, R * D :]


def _fused(parts, hroot, hin, w, g, b):
    return pl.pallas_call(
        _fused_body,
        grid=(N // BN,),
        in_specs=[
            pl.BlockSpec((NC, BN, D), lambda i: (0, i, 0)),
            pl.BlockSpec((BN, D), lambda i: (i, 0)),
            pl.BlockSpec((BN, D), lambda i: (i, 0)),
            pl.BlockSpec((D, WCOLS), lambda i: (0, 0)),
            pl.BlockSpec((1, D), lambda i: (0, 0)),
            pl.BlockSpec((1, D), lambda i: (0, 0)),
        ],
        out_specs=(
            pl.BlockSpec((BN, D), lambda i: (i, 0)),
            pl.BlockSpec((BN, R * D), lambda i: (i, 0)),
            pl.BlockSpec((BN, D), lambda i: (i, 0)),
        ),
        out_shape=(
            jax.ShapeDtypeStruct((N, D), jnp.float32),
            jax.ShapeDtypeStruct((N, R * D), jnp.float32),
            jax.ShapeDtypeStruct((N, D), jnp.float32),
        ),
    )(parts, hroot, hin, w, g, b)


def _zfin_body(part_ref, hroot_ref, hin_ref, g_ref, b_ref, z_ref):
    agg = part_ref[0] + part_ref[1] + hroot_ref[...]
    z_ref[...] = _ln(agg, g_ref[...], b_ref[...]) + hin_ref[...]


def _zfin(parts, hroot, hin, g, b):
    return pl.pallas_call(
        _zfin_body,
        grid=(N // BN,),
        in_specs=[
            pl.BlockSpec((NC, BN, D), lambda i: (0, i, 0)),
            pl.BlockSpec((BN, D), lambda i: (i, 0)),
            pl.BlockSpec((BN, D), lambda i: (i, 0)),
            pl.BlockSpec((1, D), lambda i: (0, 0)),
            pl.BlockSpec((1, D), lambda i: (0, 0)),
        ],
        out_specs=pl.BlockSpec((BN, D), lambda i: (i, 0)),
        out_shape=jax.ShapeDtypeStruct((N, D), jnp.float32),
    )(parts, hroot, hin, g, b)


@functools.partial(
    pl.kernel,
    out_type=jax.ShapeDtypeStruct((NP, NC, NS, SPT, D), jnp.float32),
    mesh=_MESH,
    scratch_types=[
        pltpu.VMEM((NCHUNK, CH), jnp.int32),
        pltpu.VMEM((NCHUNK, CH), jnp.int32),
        pltpu.VMEM((NCHUNK, CH), jnp.int32),
        pltpu.VMEM((NCHUNK, CH), jnp.int32),
        pltpu.VMEM((NCHUNK, CH), jnp.int32),
        pltpu.VMEM((CH, D), jnp.float32),
        pltpu.VMEM((CH, D), jnp.float32),
        pltpu.VMEM_SHARED((PR + GR, D), jnp.float32),
        pltpu.SemaphoreType.DMA,
        pltpu.SemaphoreType.DMA,
        pltpu.SemaphoreType.DMA,
        pltpu.SemaphoreType.DMA,
    ],
)
def _edge_kernel(t_hbm, src_hbm, et_hbm, dst_hbm, zero_hbm, out_hbm,
                 src_t, et_t, dst_t, gidx_t, ldst_t, rows0, rows1, acc,
                 sg0, sg1, ss0, ss1):
    c = lax.axis_index("c")
    s = lax.axis_index("s")
    wid = s * NC + c

    pltpu.sync_copy(zero_hbm, acc.at[pl.ds(s * SPT, SPT)])
    @pl.when(s == 0)
    def _zero_garbage():
        pltpu.sync_copy(zero_hbm.at[pl.ds(0, GR)], acc.at[pl.ds(PR, GR)])

    pltpu.sync_copy(src_hbm.at[wid], src_t)
    pltpu.sync_copy(et_hbm.at[wid], et_t)
    pltpu.sync_copy(dst_hbm.at[wid], dst_t)

    def _mk_idx(i, carry):
        for j in range(CH // 16):
            sl = pl.ds(j * 16, 16)
            gidx_t[i, sl] = src_t[i, sl] * R + et_t[i, sl]
        return carry

    lax.fori_loop(0, NCHUNK, _mk_idx, 0)

    for p in range(NP):
        def _mk_ldst(i, carry):
            for j in range(CH // 16):
                sl = pl.ds(j * 16, 16)
                dv = dst_t[i, sl]
                loc = dv - (p * PR)
                valid = (loc >= 0) & (loc < PR)
                ldst_t[i, sl] = jnp.where(valid, loc, PR + (dv & (GR - 1)))
            return carry

        lax.fori_loop(0, NCHUNK, _mk_ldst, 0)
        plsc.subcore_barrier()

        def _gather(ci, buf, sem):
            pltpu.async_copy(t_hbm.at[gidx_t.at[ci]], buf, sem)

        def _gather_wait(ci, buf, sem):
            pltpu.make_async_copy(t_hbm.at[gidx_t.at[ci]], buf, sem).wait()

        def _scatter(ci, buf, sem):
            pltpu.async_copy(buf, acc.at[ldst_t.at[ci]], sem, add=True)

        def _scatter_wait(ci, buf, sem):
            pltpu.make_async_copy(buf, acc.at[ldst_t.at[ci]], sem).wait()

        _gather(0, rows0, sg0)
        _gather(1, rows1, sg1)

        def _edge_pair(k, carry):
            c0 = 2 * k
            c1 = c0 + 1
            _gather_wait(c0, rows0, sg0)
            _scatter(c0, rows0, ss0)
            _gather_wait(c1, rows1, sg1)
            _scatter(c1, rows1, ss1)

            @pl.when(k < (NCHUNK - 1) // 2 - 1)
            def _prefetch():
                _scatter_wait(c0, rows0, ss0)
                _gather(c0 + 2, rows0, sg0)
                _scatter_wait(c1, rows1, ss1)
                _gather(c1 + 2, rows1, sg1)

            return carry

        lax.fori_loop(0, (NCHUNK - 1) // 2, _edge_pair, 0)
        _scatter_wait(NCHUNK - 3, rows0, ss0)
        _gather(NCHUNK - 1, rows0, sg0)
        _scatter_wait(NCHUNK - 2, rows1, ss1)
        _gather_wait(NCHUNK - 1, rows0, sg0)
        _scatter(NCHUNK - 1, rows0, ss0)
        _scatter_wait(NCHUNK - 1, rows0, ss0)

        plsc.subcore_barrier()
        pltpu.sync_copy(acc.at[pl.ds(s * SPT, SPT)], out_hbm.at[p, c, s])
        if p < NP - 1:
            pltpu.sync_copy(zero_hbm, acc.at[pl.ds(s * SPT, SPT)])


@functools.partial(
    pl.kernel,
    out_type=(
        jax.ShapeDtypeStruct((P, D), jnp.float32),
        jax.ShapeDtypeStruct((P, D), jnp.float32),
    ),
    mesh=_MESH,
    scratch_types=[
        pltpu.VMEM((2, 128), jnp.int32),
        pltpu.VMEM((2, 128), jnp.int32),
        pltpu.VMEM((128, D), jnp.float32),
        pltpu.SemaphoreType.DMA,
    ],
)
def _pair_kernel(z_hbm, gpi_hbm, gpj_hbm, zi_hbm, zj_hbm, ii_t, jj_t, rows, sem):
    c = lax.axis_index("c")
    s = lax.axis_index("s")
    wid = s * NC + c
    pltpu.sync_copy(gpi_hbm.at[wid], ii_t)
    pltpu.sync_copy(gpj_hbm.at[wid], jj_t)
    for k in range(2):
        row = wid * 2 + k
        pltpu.async_copy(z_hbm.at[ii_t.at[k]], rows, sem).wait()
        pltpu.sync_copy(rows, zi_hbm.at[pl.ds(row * 128, 128)])
        pltpu.async_copy(z_hbm.at[jj_t.at[k]], rows, sem).wait()
        pltpu.sync_copy(rows, zj_hbm.at[pl.ds(row * 128, 128)])


def _dec_body(zi_ref, zj_ref,
              w1_ref, b1_ref, g1_ref, e1_ref,
              w2_ref, b2_ref, g2_ref, e2_ref,
              w3_ref, b3_ref, g3_ref, e3_ref,
              w4_ref, b4_ref, o_ref):
    zi = zi_ref[...]
    zj = zj_ref[...]
    a = jnp.concatenate([zi + zj, zi * zj, jnp.abs(zi - zj)], axis=-1)
    a = jnp.dot(a, w1_ref[...], preferred_element_type=jnp.float32) + b1_ref[...]
    a = _gelu(_ln(a, g1_ref[...], e1_ref[...]))
    a = jnp.dot(a, w2_ref[...], preferred_element_type=jnp.float32) + b2_ref[...]
    a = _gelu(_ln(a, g2_ref[...], e2_ref[...]))
    a = jnp.dot(a, w3_ref[...], preferred_element_type=jnp.float32) + b3_ref[...]
    a = _gelu(_ln(a, g3_ref[...], e3_ref[...]))
    a = jnp.dot(a, w4_ref[...], preferred_element_type=jnp.float32) + b4_ref[...]
    o_ref[...] = a


def _decode(zi, zj, dec):
    full = lambda shape: pl.BlockSpec(shape, lambda i: (0, 0))
    args = [zi, zj]
    in_specs = [
        pl.BlockSpec((BP, D), lambda i: (i, 0)),
        pl.BlockSpec((BP, D), lambda i: (i, 0)),
    ]
    for li, d in enumerate(dec):
        w = d["w"]
        args += [w, d["b"].reshape(1, -1)]
        in_specs += [full(w.shape), full((1, w.shape[1]))]
        if li < 3:
            args += [d["g"].reshape(1, -1), d["beta"].reshape(1, -1)]
            in_specs += [full((1, w.shape[1])), full((1, w.shape[1]))]
    return pl.pallas_call(
        _dec_body,
        grid=(P // BP,),
        in_specs=in_specs,
        out_specs=pl.BlockSpec((BP, 1), lambda i: (i, 0)),
        out_shape=jax.ShapeDtypeStruct((P, 1), jnp.float32),
    )(*args)


def kernel(x, params, edge_index, edge_type, gene_pairs):
    layers = params["layers"]
    att = jnp.stack([l["att"] for l in layers])
    basis = jnp.stack([l["basis"] for l in layers])
    root = jnp.stack([l["root"] for l in layers])
    rel = jnp.stack([l["rel_att"] for l in layers])
    wcat = _make_wcat(att, basis, root, rel)

    npad = NW * EWP - E
    pad0 = jnp.zeros((npad,), jnp.int32)
    padd = jnp.full((npad,), NPAD, jnp.int32)
    src2 = jnp.concatenate([edge_index[0], pad0]).reshape(NW, NCHUNK, CH)
    dst2 = jnp.concatenate([edge_index[1], padd]).reshape(NW, NCHUNK, CH)
    et2 = jnp.concatenate([edge_type, pad0]).reshape(NW, NCHUNK, CH)
    zeros = jnp.zeros((SPT, D), jnp.float32)

    h = x
    hroot = None
    parts = None
    for i in range(NUM_LAYERS):
        if i == 0:
            t, hroot = _mm(h, wcat[0])
        else:
            lp = layers[i - 1]
            h, t, hroot = _fused(
                parts, hroot, h, wcat[i],
                lp["ln_g"].reshape(1, D), lp["ln_b"].reshape(1, D),
            )
        table = t.reshape(N * R, D)
        po = _edge_kernel(table, src2, et2, dst2, zeros)
        parts = (
            po.reshape(NP, NC, PR, D)
            .transpose(1, 0, 2, 3)
            .reshape(NC, NPAD, D)[:, :N, :]
        )

    lp = layers[NUM_LAYERS - 1]
    z = _zfin(parts, hroot, h,
              lp["ln_g"].reshape(1, D), lp["ln_b"].reshape(1, D))

    gpi = gene_pairs[0].reshape(NW, 2, 128)
    gpj = gene_pairs[1].reshape(NW, 2, 128)
    zi, zj = _pair_kernel(z, gpi, gpj)

    pred = _decode(zi, zj, params["dec"]).reshape(P)
    return (pred, z)


# speedup vs baseline: 1.7969x; 1.7969x over previous
import functools

import jax
import jax.numpy as jnp
from jax import lax
from jax.experimental import pallas as pl
from jax.experimental.pallas import tpu as pltpu
from jax.experimental.pallas import tpu_sc as plsc

N = 10000
E = 320000
R = 8
B = 4
D = 128
NUM_LAYERS = 4
P = 8192

NC = 2
NS = 16
NW = NC * NS
EW = E // NW
CH = 80
NCHUNK = EW // CH
RPT = N // NS
BN = 400
BP = 512
WCOLS = (R + 1) * D
NP = 3
PR = 3456
SPT = PR // NS
GR = 128
NPAD = NP * PR

_MESH = plsc.VectorSubcoreMesh(
    core_axis_name="c", subcore_axis_name="s", num_cores=NC, num_subcores=NS
)

_SQRT_HALF = 0.7071067811865476


def _gelu(x):
    return 0.5 * x * (1.0 + lax.erf(x * _SQRT_HALF))


def _ln(h, g, b, eps=1e-5):
    mu = jnp.mean(h, axis=-1, keepdims=True)
    var = jnp.mean((h - mu) ** 2, axis=-1, keepdims=True)
    return g * (h - mu) / jnp.sqrt(var + eps) + b


def _bf16_round(x):
    return x.astype(jnp.bfloat16).astype(jnp.float32)


def _wcat_body(att_ref, basis_ref, root_ref, rel_ref, out_ref):
    att = _bf16_round(att_ref[...])
    basis = _bf16_round(basis_ref[...])
    root = root_ref[...]
    rel = rel_ref[...]
    for i in range(NUM_LAYERS):
        relm = rel[i : i + 1, :]
        ex = jnp.exp(relm - jnp.max(relm, axis=-1, keepdims=True))
        attn = ex / jnp.sum(ex, axis=-1, keepdims=True)
        for r in range(R):
            w = None
            for b in range(B):
                srb = att[i, r : r + 1, b : b + 1] * attn[:, r : r + 1]
                term = srb * basis[i, b]
                w = term if w is None else w + term
            out_ref[i, :, r * D : (r + 1) * D] = w
        out_ref[i, :, R * D :] = root[i]


def _make_wcat(att, basis, root, rel):
    return pl.pallas_call(
        _wcat_body,
        out_shape=jax.ShapeDtypeStruct((NUM_LAYERS, D, WCOLS), jnp.float32),
    )(att, basis, root, rel)


def _mm_body(h_ref, w_ref, t_ref, r_ref):
    a = jnp.dot(h_ref[...], w_ref[...], preferred_element_type=jnp.float32)
    t_ref[...] = a[:, : R * D]
    r_ref[...] = a[:, R * D :]


def _mm(h, w):
    return pl.pallas_call(
        _mm_body,
        grid=(N // BN,),
        in_specs=[
            pl.BlockSpec((BN, D), lambda i: (i, 0)),
            pl.BlockSpec((D, WCOLS), lambda i: (0, 0)),
        ],
        out_specs=(
            pl.BlockSpec((BN, R * D), lambda i: (i, 0)),
            pl.BlockSpec((BN, D), lambda i: (i, 0)),
        ),
        out_shape=(
            jax.ShapeDtypeStruct((N, R * D), jnp.float32),
            jax.ShapeDtypeStruct((N, D), jnp.float32),
        ),
    )(h, w)


def _fused_body(part_ref, hroot_ref, hin_ref, w_ref, g_ref, b_ref,
                h_ref, t_ref, r_ref):
    agg = part_ref[0] + part_ref[1] + hroot_ref[...]
    h = _ln(agg, g_ref[...], b_ref[...]) + hin_ref[...]
    h = _gelu(h)
    h_ref[...] = h
    a = jnp.dot(h, w_ref[...], preferred_element_type=jnp.float32)
    t_ref[...] = a[:, : R * D]
    r_ref[...] = a[:, R * D :]


def _fused(parts, hroot, hin, w, g, b):
    return pl.pallas_call(
        _fused_body,
        grid=(N // BN,),
        in_specs=[
            pl.BlockSpec((NC, BN, D), lambda i: (0, i, 0)),
            pl.BlockSpec((BN, D), lambda i: (i, 0)),
            pl.BlockSpec((BN, D), lambda i: (i, 0)),
            pl.BlockSpec((D, WCOLS), lambda i: (0, 0)),
            pl.BlockSpec((1, D), lambda i: (0, 0)),
            pl.BlockSpec((1, D), lambda i: (0, 0)),
        ],
        out_specs=(
            pl.BlockSpec((BN, D), lambda i: (i, 0)),
            pl.BlockSpec((BN, R * D), lambda i: (i, 0)),
            pl.BlockSpec((BN, D), lambda i: (i, 0)),
        ),
        out_shape=(
            jax.ShapeDtypeStruct((N, D), jnp.float32),
            jax.ShapeDtypeStruct((N, R * D), jnp.float32),
            jax.ShapeDtypeStruct((N, D), jnp.float32),
        ),
    )(parts, hroot, hin, w, g, b)


def _zfin_body(part_ref, hroot_ref, hin_ref, g_ref, b_ref, z_ref):
    agg = part_ref[0] + part_ref[1] + hroot_ref[...]
    z_ref[...] = _ln(agg, g_ref[...], b_ref[...]) + hin_ref[...]


def _zfin(parts, hroot, hin, g, b):
    return pl.pallas_call(
        _zfin_body,
        grid=(N // BN,),
        in_specs=[
            pl.BlockSpec((NC, BN, D), lambda i: (0, i, 0)),
            pl.BlockSpec((BN, D), lambda i: (i, 0)),
            pl.BlockSpec((BN, D), lambda i: (i, 0)),
            pl.BlockSpec((1, D), lambda i: (0, 0)),
            pl.BlockSpec((1, D), lambda i: (0, 0)),
        ],
        out_specs=pl.BlockSpec((BN, D), lambda i: (i, 0)),
        out_shape=jax.ShapeDtypeStruct((N, D), jnp.float32),
    )(parts, hroot, hin, g, b)


@functools.partial(
    pl.kernel,
    out_type=jax.ShapeDtypeStruct((NP, NC, NS, SPT, D), jnp.float32),
    mesh=_MESH,
    scratch_types=[
        pltpu.VMEM((NCHUNK, CH), jnp.int32),
        pltpu.VMEM((NCHUNK, CH), jnp.int32),
        pltpu.VMEM((NCHUNK, CH), jnp.int32),
        pltpu.VMEM((NCHUNK, CH), jnp.int32),
        pltpu.VMEM((NCHUNK, CH), jnp.int32),
        pltpu.VMEM((CH, D), jnp.float32),
        pltpu.VMEM((CH, D), jnp.float32),
        pltpu.VMEM_SHARED((PR + GR, D), jnp.float32),
        pltpu.SemaphoreType.DMA,
        pltpu.SemaphoreType.DMA,
        pltpu.SemaphoreType.DMA,
        pltpu.SemaphoreType.DMA,
    ],
)
def _edge_kernel(t_hbm, src_hbm, et_hbm, dst_hbm, zero_hbm, out_hbm,
                 src_t, et_t, dst_t, gidx_t, ldst_t, rows0, rows1, acc,
                 sg0, sg1, ss0, ss1):
    c = lax.axis_index("c")
    s = lax.axis_index("s")
    wid = s * NC + c

    pltpu.sync_copy(zero_hbm, acc.at[pl.ds(s * SPT, SPT)])
    @pl.when(s == 0)
    def _zero_garbage():
        pltpu.sync_copy(zero_hbm.at[pl.ds(0, GR)], acc.at[pl.ds(PR, GR)])

    pltpu.sync_copy(src_hbm.at[wid], src_t)
    pltpu.sync_copy(et_hbm.at[wid], et_t)
    pltpu.sync_copy(dst_hbm.at[wid], dst_t)

    def _mk_idx(i, carry):
        for j in range(CH // 16):
            sl = pl.ds(j * 16, 16)
            gidx_t[i, sl] = src_t[i, sl] * R + et_t[i, sl]
        return carry

    lax.fori_loop(0, NCHUNK, _mk_idx, 0)

    for p in range(NP):
        def _mk_ldst(i, carry):
            for j in range(CH // 16):
                sl = pl.ds(j * 16, 16)
                dv = dst_t[i, sl]
                loc = dv - (p * PR)
                valid = (loc >= 0) & (loc < PR)
                ldst_t[i, sl] = jnp.where(valid, loc, PR + (dv & (GR - 1)))
            return carry

        lax.fori_loop(0, NCHUNK, _mk_ldst, 0)
        plsc.subcore_barrier()

        def _gather(ci, buf, sem):
            pltpu.async_copy(t_hbm.at[gidx_t.at[ci]], buf, sem)

        def _gather_wait(ci, buf, sem):
            pltpu.make_async_copy(t_hbm.at[gidx_t.at[ci]], buf, sem).wait()

        def _scatter(ci, buf, sem):
            pltpu.async_copy(buf, acc.at[ldst_t.at[ci]], sem, add=True)

        def _scatter_wait(ci, buf, sem):
            pltpu.make_async_copy(buf, acc.at[ldst_t.at[ci]], sem).wait()

        _gather(0, rows0, sg0)
        _gather(1, rows1, sg1)

        def _edge_pair(k, carry):
            c0 = 2 * k
            c1 = c0 + 1
            _gather_wait(c0, rows0, sg0)
            _scatter(c0, rows0, ss0)
            _gather_wait(c1, rows1, sg1)
            _scatter(c1, rows1, ss1)

            @pl.when(k < (NCHUNK - 1) // 2 - 1)
            def _prefetch():
                _scatter_wait(c0, rows0, ss0)
                _gather(c0 + 2, rows0, sg0)
                _scatter_wait(c1, rows1, ss1)
                _gather(c1 + 2, rows1, sg1)

            return carry

        lax.fori_loop(0, (NCHUNK - 1) // 2, _edge_pair, 0)
        _scatter_wait(NCHUNK - 3, rows0, ss0)
        _gather(NCHUNK - 1, rows0, sg0)
        _scatter_wait(NCHUNK - 2, rows1, ss1)
        _gather_wait(NCHUNK - 1, rows0, sg0)
        _scatter(NCHUNK - 1, rows0, ss0)
        _scatter_wait(NCHUNK - 1, rows0, ss0)

        plsc.subcore_barrier()
        pltpu.sync_copy(acc.at[pl.ds(s * SPT, SPT)], out_hbm.at[p, c, s])
        if p < NP - 1:
            pltpu.sync_copy(zero_hbm, acc.at[pl.ds(s * SPT, SPT)])


@functools.partial(
    pl.kernel,
    out_type=(
        jax.ShapeDtypeStruct((P, D), jnp.float32),
        jax.ShapeDtypeStruct((P, D), jnp.float32),
    ),
    mesh=_MESH,
    scratch_types=[
        pltpu.VMEM((2, 128), jnp.int32),
        pltpu.VMEM((2, 128), jnp.int32),
        pltpu.VMEM((128, D), jnp.float32),
        pltpu.SemaphoreType.DMA,
    ],
)
def _pair_kernel(z_hbm, gpi_hbm, gpj_hbm, zi_hbm, zj_hbm, ii_t, jj_t, rows, sem):
    c = lax.axis_index("c")
    s = lax.axis_index("s")
    wid = s * NC + c
    pltpu.sync_copy(gpi_hbm.at[wid], ii_t)
    pltpu.sync_copy(gpj_hbm.at[wid], jj_t)
    for k in range(2):
        row = wid * 2 + k
        pltpu.async_copy(z_hbm.at[ii_t.at[k]], rows, sem).wait()
        pltpu.sync_copy(rows, zi_hbm.at[pl.ds(row * 128, 128)])
        pltpu.async_copy(z_hbm.at[jj_t.at[k]], rows, sem).wait()
        pltpu.sync_copy(rows, zj_hbm.at[pl.ds(row * 128, 128)])


def _dec_body(zi_ref, zj_ref,
              w1_ref, b1_ref, g1_ref, e1_ref,
              w2_ref, b2_ref, g2_ref, e2_ref,
              w3_ref, b3_ref, g3_ref, e3_ref,
              w4_ref, b4_ref, o_ref):
    zi = zi_ref[...]
    zj = zj_ref[...]
    a = jnp.concatenate([zi + zj, zi * zj, jnp.abs(zi - zj)], axis=-1)
    a = jnp.dot(a, w1_ref[...], preferred_element_type=jnp.float32) + b1_ref[...]
    a = _gelu(_ln(a, g1_ref[...], e1_ref[...]))
    a = jnp.dot(a, w2_ref[...], preferred_element_type=jnp.float32) + b2_ref[...]
    a = _gelu(_ln(a, g2_ref[...], e2_ref[...]))
    a = jnp.dot(a, w3_ref[...], preferred_element_type=jnp.float32) + b3_ref[...]
    a = _gelu(_ln(a, g3_ref[...], e3_ref[...]))
    a = jnp.dot(a, w4_ref[...], preferred_element_type=jnp.float32) + b4_ref[...]
    o_ref[...] = a


def _decode(zi, zj, dec):
    full = lambda shape: pl.BlockSpec(shape, lambda i: (0, 0))
    args = [zi, zj]
    in_specs = [
        pl.BlockSpec((BP, D), lambda i: (i, 0)),
        pl.BlockSpec((BP, D), lambda i: (i, 0)),
    ]
    for li, d in enumerate(dec):
        w = d["w"]
        args += [w, d["b"].reshape(1, -1)]
        in_specs += [full(w.shape), full((1, w.shape[1]))]
        if li < 3:
            args += [d["g"].reshape(1, -1), d["beta"].reshape(1, -1)]
            in_specs += [full((1, w.shape[1])), full((1, w.shape[1]))]
    return pl.pallas_call(
        _dec_body,
        grid=(P // BP,),
        in_specs=in_specs,
        out_specs=pl.BlockSpec((BP, 1), lambda i: (i, 0)),
        out_shape=jax.ShapeDtypeStruct((P, 1), jnp.float32),
    )(*args)


def kernel(x, params, edge_index, edge_type, gene_pairs):
    layers = params["layers"]
    att = jnp.stack([l["att"] for l in layers])
    basis = jnp.stack([l["basis"] for l in layers])
    root = jnp.stack([l["root"] for l in layers])
    rel = jnp.stack([l["rel_att"] for l in layers])
    wcat = _make_wcat(att, basis, root, rel)

    src2 = edge_index[0].reshape(NW, NCHUNK, CH)
    dst2 = edge_index[1].reshape(NW, NCHUNK, CH)
    et2 = edge_type.reshape(NW, NCHUNK, CH)
    zeros = jnp.zeros((SPT, D), jnp.float32)

    h = x
    hroot = None
    parts = None
    for i in range(NUM_LAYERS):
        if i == 0:
            t, hroot = _mm(h, wcat[0])
        else:
            lp = layers[i - 1]
            h, t, hroot = _fused(
                parts, hroot, h, wcat[i],
                lp["ln_g"].reshape(1, D), lp["ln_b"].reshape(1, D),
            )
        table = t.reshape(N * R, D)
        po = _edge_kernel(table, src2, et2, dst2, zeros)
        parts = (
            po.reshape(NP, NC, PR, D)
            .transpose(1, 0, 2, 3)
            .reshape(NC, NPAD, D)[:, :N, :]
        )

    lp = layers[NUM_LAYERS - 1]
    z = _zfin(parts, hroot, h,
              lp["ln_g"].reshape(1, D), lp["ln_b"].reshape(1, D))

    gpi = gene_pairs[0].reshape(NW, 2, 128)
    gpj = gene_pairs[1].reshape(NW, 2, 128)
    zi, zj = _pair_kernel(z, gpi, gpj)

    pred = _decode(zi, zj, params["dec"]).reshape(P)
    return (pred, z)
